# Initial kernel scaffold; baseline (speedup 1.0000x reference)
import jax
import jax.numpy as jnp
from jax.experimental import pallas as pl


def kernel(features0, features1, features2, features3, features4, mesh_coords, mesh_features):
    raise NotImplementedError("write your pallas kernel here")


# speedup vs baseline: 8.5659x; 8.5659x over previous
import functools

import jax
import jax.numpy as jnp
import numpy as np
from jax import lax
from jax.experimental import pallas as pl
from jax.experimental.pallas import tpu as pltpu
from jax.experimental.pallas import tpu_sc as plsc

_NC = 2
_NS = 16
_NW = _NC * _NS
_P = 16384
_PPW = _P // _NW
_CH = 16
_NCHUNK = _PPW // _CH

_LEVELS = ((32, 64), (16, 128), (8, 256))


def _axis_setup(c_ref, off, factor, hi):
  s = c_ref[pl.ds(off, _CH)] * factor
  s = jnp.minimum(jnp.maximum(s, np.float32(0.01)), np.float32(hi))
  i1 = s.astype(jnp.int32)
  f1 = i1.astype(jnp.float32)
  frac = s - f1
  i2 = i1 + jnp.where(frac > np.float32(0.0), 1, 0).astype(jnp.int32)
  w_lo = i2.astype(jnp.float32) - s
  w_hi = frac
  return i1, i2, w_lo, w_hi


def _sc_body(t1, t2, t3, cx, cy, cz, o1, o2, o3,
             cx_v, cy_v, cz_v, idx_v,
             rows1, rows2, rows3, outv1, outv2, outv3, sem):
  wid = lax.axis_index("s") * _NC + lax.axis_index("c")
  base = wid * _PPW

  pltpu.sync_copy(cx.at[pl.ds(base, _PPW)], cx_v)
  pltpu.sync_copy(cy.at[pl.ds(base, _PPW)], cy_v)
  pltpu.sync_copy(cz.at[pl.ds(base, _PPW)], cz_v)

  for (size, ch), t_ref, rows_ref, outv_ref, o_ref in zip(
      _LEVELS, (t1, t2, t3), (rows1, rows2, rows3),
      (outv1, outv2, outv3), (o1, o2, o3)):
    factor = float(size)
    hi = size - 1.01
    nvec = ch // 16

    @pl.loop(0, _NCHUNK)
    def _chunk(i, t_ref=t_ref, rows_ref=rows_ref, outv_ref=outv_ref,
               o_ref=o_ref, size=size, ch=ch, factor=factor, hi=hi,
               nvec=nvec):
      off = i * _CH
      x1, x2, wx1, wx2 = _axis_setup(cx_v, off, factor, hi)
      y1, y2, wy1, wy2 = _axis_setup(cy_v, off, factor, hi)
      z1, _, wz1, wz2 = _axis_setup(cz_v, off, factor, hi)

      s32 = np.int32(size)
      zdim = np.int32(size - 1)
      wlo = []
      whi = []
      for kp, (xa, wxa, yb, wyb) in enumerate((
          (x1, wx1, y1, wy1), (x1, wx1, y2, wy2),
          (x2, wx2, y1, wy1), (x2, wx2, y2, wy2))):
        idx_v[pl.ds(kp * _CH, _CH)] = (xa * s32 + yb) * zdim + z1
        wxy = wxa * wyb
        wlo.append(wxy * wz1)
        whi.append(wxy * wz2)

      pltpu.async_copy(t_ref.at[idx_v], rows_ref, sem).wait()

      for p in range(_CH):
        ws = [(wlo[kp][p], whi[kp][p]) for kp in range(4)]

        @pl.loop(0, nvec)
        def _chanvec(j, p=p, ws=ws, ch=ch, rows_ref=rows_ref,
                     outv_ref=outv_ref):
          lo = pl.ds(j * 16, 16)
          hi_sl = pl.ds(ch + j * 16, 16)
          acc = ws[0][0] * rows_ref[p, lo] + ws[0][1] * rows_ref[p, hi_sl]
          for kp in range(1, 4):
            row = kp * _CH + p
            acc = (acc + ws[kp][0] * rows_ref[row, lo]
                   + ws[kp][1] * rows_ref[row, hi_sl])
          outv_ref[p, lo] = acc

      pltpu.sync_copy(outv_ref, o_ref.at[pl.ds(base + off, _CH)])


@jax.jit
def _projection_sc(t1, t2, t3, cx, cy, cz):
  mesh = plsc.VectorSubcoreMesh(core_axis_name="c", subcore_axis_name="s")
  out_type = (
      jax.ShapeDtypeStruct((_P, 64), jnp.float32),
      jax.ShapeDtypeStruct((_P, 128), jnp.float32),
      jax.ShapeDtypeStruct((_P, 256), jnp.float32),
  )
  scratch = [
      pltpu.VMEM((_PPW,), jnp.float32),
      pltpu.VMEM((_PPW,), jnp.float32),
      pltpu.VMEM((_PPW,), jnp.float32),
      pltpu.VMEM((4 * _CH,), jnp.int32),
      pltpu.VMEM((4 * _CH, 128), jnp.float32),
      pltpu.VMEM((4 * _CH, 256), jnp.float32),
      pltpu.VMEM((4 * _CH, 512), jnp.float32),
      pltpu.VMEM((_CH, 64), jnp.float32),
      pltpu.VMEM((_CH, 128), jnp.float32),
      pltpu.VMEM((_CH, 256), jnp.float32),
      pltpu.SemaphoreType.DMA,
  ]
  run = pl.kernel(_sc_body, out_type=out_type, mesh=mesh,
                  scratch_types=scratch)
  return run(t1, t2, t3, cx, cy, cz)


def _zfuse(f, size, ch):
  x = f[0]
  fused = jnp.concatenate([x[:, :, :-1, :], x[:, :, 1:, :]], axis=-1)
  return fused.reshape(size * size * (size - 1), 2 * ch)


def kernel(features0, features1, features2, features3, features4,
           mesh_coords, mesh_features):
  t1 = _zfuse(features1, 32, 64)
  t2 = _zfuse(features2, 16, 128)
  t3 = _zfuse(features3, 8, 256)
  mc = mesh_coords[0]
  o1, o2, o3 = _projection_sc(t1, t2, t3, mc[:, 0], mc[:, 1], mc[:, 2])
  out = jnp.concatenate([o1, o2, o3, mesh_features[0]], axis=-1)
  return out[None]


# speedup vs baseline: 10.6677x; 1.2454x over previous
import functools

import jax
import jax.numpy as jnp
import numpy as np
from jax import lax
from jax.experimental import pallas as pl
from jax.experimental.pallas import tpu as pltpu
from jax.experimental.pallas import tpu_sc as plsc

_NC = 2
_NS = 16
_NW = _NC * _NS
_P = 16384
_PPW = _P // _NW
_CH = 16
_NCHUNK = _PPW // _CH

_LEVELS = ((32, 64), (16, 128), (8, 256))
_UNROLL = (4, 4, 4)


def _axis_setup(c_ref, off, size):
  s = c_ref[pl.ds(off, _CH)] * np.float32(size)
  s = jnp.minimum(jnp.maximum(s, np.float32(0.01)), np.float32(size - 1.01))
  i1 = s.astype(jnp.int32)
  f1 = i1.astype(jnp.float32)
  frac = s - f1
  i2 = i1 + jnp.where(frac > np.float32(0.0), 1, 0).astype(jnp.int32)
  w_lo = i2.astype(jnp.float32) - s
  w_hi = frac
  return i1, w_lo, w_hi


_DNUMS = lax.GatherDimensionNumbers(
    offset_dims=(), collapsed_slice_dims=(0,), start_index_map=(0,))


def _wbcast(w, p):
  idx = jnp.full((16,), p, jnp.int32)
  return lax.gather(w, idx[:, None], dimension_numbers=_DNUMS,
                    slice_sizes=(1,),
                    mode=lax.GatherScatterMode.PROMISE_IN_BOUNDS)


def _corner_weights(cx_v, cy_v, cz_v, off, size):
  x1, wx1, wx2 = _axis_setup(cx_v, off, size)
  y1, wy1, wy2 = _axis_setup(cy_v, off, size)
  z1, wz1, wz2 = _axis_setup(cz_v, off, size)
  x2i = x1 + 1
  y2i = y1 + 1
  pairs = ((x1, wx1, y1, wy1), (x1, wx1, y2i, wy2),
           (x2i, wx2, y1, wy1), (x2i, wx2, y2i, wy2))
  return z1, wz1, wz2, pairs


def _sc_body(t1, t2, t3, cx, cy, cz, o1, o2, o3,
             cx_v, cy_v, cz_v, idxA, idxB,
             rA1, rA2, rA3, rB1, rB2, rB3,
             outv1, outv2, outv3, semA, semB):
  wid = lax.axis_index("s") * _NC + lax.axis_index("c")
  base = wid * _PPW

  pltpu.sync_copy(cx.at[pl.ds(base, _PPW)], cx_v)
  pltpu.sync_copy(cy.at[pl.ds(base, _PPW)], cy_v)
  pltpu.sync_copy(cz.at[pl.ds(base, _PPW)], cz_v)

  tabs = (t1, t2, t3)
  outs = (o1, o2, o3)
  outvs = (outv1, outv2, outv3)
  bufs = {"A": (idxA, (rA1, rA2, rA3), semA),
          "B": (idxB, (rB1, rB2, rB3), semB)}

  def fire(i, which):
    idx_ref, rows, sem = bufs[which]
    off = i * _CH
    for li, (size, ch) in enumerate(_LEVELS):
      z1, _, _, pairs = _corner_weights(cx_v, cy_v, cz_v, off, size)
      s32 = np.int32(size)
      zdim = np.int32(size - 1)
      for kp, (xa, _, yb, _) in enumerate(pairs):
        idx_ref[pl.ds(li * 64 + kp * _CH, _CH)] = (xa * s32 + yb) * zdim + z1
    for li in range(3):
      pltpu.async_copy(tabs[li].at[idx_ref.at[pl.ds(li * 64, 64)]],
                       rows[li], sem)

  def wait_compute(i, which):
    idx_ref, rows, sem = bufs[which]
    for li in range(3):
      pltpu.make_async_copy(tabs[li].at[pl.ds(0, 64)], rows[li], sem).wait()
    off = i * _CH
    for li, (size, ch) in enumerate(_LEVELS):
      _, wz1, wz2, pairs = _corner_weights(cx_v, cy_v, cz_v, off, size)
      wlo = [wxa * wyb * wz1 for (_, wxa, _, wyb) in pairs]
      whi = [wxa * wyb * wz2 for (_, wxa, _, wyb) in pairs]
      rows_ref = rows[li]
      outv_ref = outvs[li]
      nvec = ch // 16

      @pl.loop(0, _CH)
      def _point(p, wlo=wlo, whi=whi, rows_ref=rows_ref,
                 outv_ref=outv_ref, ch=ch, nvec=nvec, li=li):
        ws = ([_wbcast(w, p) for w in wlo], [_wbcast(w, p) for w in whi])

        @pl.loop(0, nvec, unroll=_UNROLL[li])
        def _chanvec(j, p=p, ws=ws, ch=ch, rows_ref=rows_ref,
                     outv_ref=outv_ref):
          lo = pl.ds(j * 16, 16)
          hi = pl.ds(ch + j * 16, 16)
          acc = ws[0][0] * rows_ref[p, lo] + ws[1][0] * rows_ref[p, hi]
          for kp in range(1, 4):
            row = kp * _CH + p
            acc = (acc + ws[0][kp] * rows_ref[row, lo]
                   + ws[1][kp] * rows_ref[row, hi])
          outv_ref[p, lo] = acc

      pltpu.sync_copy(outv_ref, outs[li].at[pl.ds(base + off, _CH)])

  fire(0, "A")

  @pl.loop(0, _NCHUNK, step=2)
  def _sched(i):
    fire(i + 1, "B")
    wait_compute(i, "A")

    @pl.when(i + 2 < _NCHUNK)
    def _():
      fire(i + 2, "A")

    wait_compute(i + 1, "B")


@jax.jit
def _projection_sc(t1, t2, t3, cx, cy, cz):
  mesh = plsc.VectorSubcoreMesh(core_axis_name="c", subcore_axis_name="s")
  out_type = (
      jax.ShapeDtypeStruct((_P, 64), jnp.float32),
      jax.ShapeDtypeStruct((_P, 128), jnp.float32),
      jax.ShapeDtypeStruct((_P, 256), jnp.float32),
  )
  scratch = [
      pltpu.VMEM((_PPW,), jnp.float32),
      pltpu.VMEM((_PPW,), jnp.float32),
      pltpu.VMEM((_PPW,), jnp.float32),
      pltpu.VMEM((192,), jnp.int32),
      pltpu.VMEM((192,), jnp.int32),
      pltpu.VMEM((64, 128), jnp.float32),
      pltpu.VMEM((64, 256), jnp.float32),
      pltpu.VMEM((64, 512), jnp.float32),
      pltpu.VMEM((64, 128), jnp.float32),
      pltpu.VMEM((64, 256), jnp.float32),
      pltpu.VMEM((64, 512), jnp.float32),
      pltpu.VMEM((_CH, 64), jnp.float32),
      pltpu.VMEM((_CH, 128), jnp.float32),
      pltpu.VMEM((_CH, 256), jnp.float32),
      pltpu.SemaphoreType.DMA,
      pltpu.SemaphoreType.DMA,
  ]
  run = pl.kernel(_sc_body, out_type=out_type, mesh=mesh,
                  scratch_types=scratch)
  return run(t1, t2, t3, cx, cy, cz)


def _zfuse(f, size, ch):
  x = f[0]
  fused = jnp.concatenate([x[:, :, :-1, :], x[:, :, 1:, :]], axis=-1)
  return fused.reshape(size * size * (size - 1), 2 * ch)


def kernel(features0, features1, features2, features3, features4,
           mesh_coords, mesh_features):
  t1 = _zfuse(features1, 32, 64)
  t2 = _zfuse(features2, 16, 128)
  t3 = _zfuse(features3, 8, 256)
  mc = mesh_coords[0]
  o1, o2, o3 = _projection_sc(t1, t2, t3, mc[:, 0], mc[:, 1], mc[:, 2])
  out = jnp.concatenate([o1, o2, o3, mesh_features[0]], axis=-1)
  return out[None]


# speedup vs baseline: 11.2573x; 1.0553x over previous
import functools

import jax
import jax.numpy as jnp
import numpy as np
from jax import lax
from jax.experimental import pallas as pl
from jax.experimental.pallas import tpu as pltpu
from jax.experimental.pallas import tpu_sc as plsc

_NC = 2
_NS = 16
_NW = _NC * _NS
_P = 16384
_PPW = _P // _NW
_CH = 16
_NCHUNK = _PPW // _CH
_NOUT = 451

_LEVELS = ((32, 64, 0), (16, 128, 64), (8, 256, 192))
_UNROLL = (4, 4, 4)


def _axis_setup(c_ref, off, size):
  s = c_ref[pl.ds(off, _CH)] * np.float32(size)
  s = jnp.minimum(jnp.maximum(s, np.float32(0.01)), np.float32(size - 1.01))
  i1 = s.astype(jnp.int32)
  f1 = i1.astype(jnp.float32)
  frac = s - f1
  i2 = i1 + jnp.where(frac > np.float32(0.0), 1, 0).astype(jnp.int32)
  w_lo = i2.astype(jnp.float32) - s
  w_hi = frac
  return i1, w_lo, w_hi


_DNUMS = lax.GatherDimensionNumbers(
    offset_dims=(), collapsed_slice_dims=(0,), start_index_map=(0,))


def _wbcast(w, p):
  idx = jnp.full((16,), p, jnp.int32)
  return lax.gather(w, idx[:, None], dimension_numbers=_DNUMS,
                    slice_sizes=(1,),
                    mode=lax.GatherScatterMode.PROMISE_IN_BOUNDS)


def _corner_setup(cx_v, cy_v, cz_v, off, size):
  x1, wx1, wx2 = _axis_setup(cx_v, off, size)
  y1, wy1, wy2 = _axis_setup(cy_v, off, size)
  z1, wz1, wz2 = _axis_setup(cz_v, off, size)
  x2i = x1 + 1
  y2i = y1 + 1
  pairs = ((x1, wx1, y1, wy1), (x1, wx1, y2i, wy2),
           (x2i, wx2, y1, wy1), (x2i, wx2, y2i, wy2))
  return z1, wz1, wz2, pairs


def _sc_body(t1, t2, t3, cx, cy, cz, mf, out,
             cx_v, cy_v, cz_v, mf_v, idxA, idxB,
             rA1, rA2, rA3, rB1, rB2, rB3, outv, semA, semB):
  wid = lax.axis_index("s") * _NC + lax.axis_index("c")
  base = wid * _PPW

  pltpu.sync_copy(cx.at[pl.ds(base, _PPW)], cx_v)
  pltpu.sync_copy(cy.at[pl.ds(base, _PPW)], cy_v)
  pltpu.sync_copy(cz.at[pl.ds(base, _PPW)], cz_v)
  pltpu.sync_copy(mf.at[pl.ds(3 * base, 3 * _PPW)], mf_v.at[pl.ds(16, 3 * _PPW)])

  tabs = (t1, t2, t3)
  bufs = {"A": (idxA, (rA1, rA2, rA3), semA),
          "B": (idxB, (rB1, rB2, rB3), semB)}

  def fire(i, which):
    idx_ref, rows, sem = bufs[which]
    off = i * _CH
    for li, (size, ch, _) in enumerate(_LEVELS):
      z1, _, _, pairs = _corner_setup(cx_v, cy_v, cz_v, off, size)
      s32 = np.int32(size)
      zdim = np.int32(size - 1)
      for kp, (xa, _, yb, _) in enumerate(pairs):
        idx_ref[pl.ds(li * 64 + kp * _CH, _CH)] = (xa * s32 + yb) * zdim + z1
    for li in range(3):
      pltpu.async_copy(tabs[li].at[idx_ref.at[pl.ds(li * 64, 64)]],
                       rows[li], sem)

  def wait_compute(i, which):
    idx_ref, rows, sem = bufs[which]
    for li in range(3):
      pltpu.make_async_copy(tabs[li].at[pl.ds(0, 64)], rows[li], sem).wait()
    off = i * _CH
    for li, (size, ch, col) in enumerate(_LEVELS):
      _, wz1, wz2, pairs = _corner_setup(cx_v, cy_v, cz_v, off, size)
      wlo = [wxa * wyb * wz1 for (_, wxa, _, wyb) in pairs]
      whi = [wxa * wyb * wz2 for (_, wxa, _, wyb) in pairs]
      rows_ref = rows[li]
      nvec = ch // 16

      @pl.loop(0, _CH)
      def _point(p, wlo=wlo, whi=whi, rows_ref=rows_ref,
                 ch=ch, col=col, nvec=nvec, li=li, off=off):
        ws = ([_wbcast(w, p) for w in wlo], [_wbcast(w, p) for w in whi])
        if li == 0:
          outv[p, pl.ds(435, 16)] = mf_v[pl.ds(3 * (off + p) + 3, 16)]

        @pl.loop(0, nvec, unroll=_UNROLL[li])
        def _chanvec(j, p=p, ws=ws, ch=ch, col=col, rows_ref=rows_ref):
          lo = pl.ds(j * 16, 16)
          hi = pl.ds(ch + j * 16, 16)
          acc = ws[0][0] * rows_ref[p, lo] + ws[1][0] * rows_ref[p, hi]
          for kp in range(1, 4):
            row = kp * _CH + p
            acc = (acc + ws[0][kp] * rows_ref[row, lo]
                   + ws[1][kp] * rows_ref[row, hi])
          outv[p, pl.ds(col + j * 16, 16)] = acc

    pltpu.sync_copy(outv, out.at[pl.ds(base + off, _CH)])

  fire(0, "A")

  @pl.loop(0, _NCHUNK, step=2)
  def _sched(i):
    fire(i + 1, "B")
    wait_compute(i, "A")

    @pl.when(i + 2 < _NCHUNK)
    def _():
      fire(i + 2, "A")

    wait_compute(i + 1, "B")


@jax.jit
def _projection_sc(t1, t2, t3, cx, cy, cz, mf):
  mesh = plsc.VectorSubcoreMesh(core_axis_name="c", subcore_axis_name="s")
  out_type = jax.ShapeDtypeStruct((_P, _NOUT), jnp.float32)
  scratch = [
      pltpu.VMEM((_PPW,), jnp.float32),
      pltpu.VMEM((_PPW,), jnp.float32),
      pltpu.VMEM((_PPW,), jnp.float32),
      pltpu.VMEM((16 + 3 * _PPW,), jnp.float32),
      pltpu.VMEM((192,), jnp.int32),
      pltpu.VMEM((192,), jnp.int32),
      pltpu.VMEM((64, 128), jnp.float32),
      pltpu.VMEM((64, 256), jnp.float32),
      pltpu.VMEM((64, 512), jnp.float32),
      pltpu.VMEM((64, 128), jnp.float32),
      pltpu.VMEM((64, 256), jnp.float32),
      pltpu.VMEM((64, 512), jnp.float32),
      pltpu.VMEM((_CH, _NOUT), jnp.float32),
      pltpu.SemaphoreType.DMA,
      pltpu.SemaphoreType.DMA,
  ]
  run = pl.kernel(_sc_body, out_type=out_type, mesh=mesh,
                  scratch_types=scratch)
  return run(t1, t2, t3, cx, cy, cz, mf)


def _zfuse(f, size, ch):
  x = f[0]
  fused = jnp.concatenate([x[:, :, :-1, :], x[:, :, 1:, :]], axis=-1)
  return fused.reshape(size * size * (size - 1), 2 * ch)


def kernel(features0, features1, features2, features3, features4,
           mesh_coords, mesh_features):
  t1 = _zfuse(features1, 32, 64)
  t2 = _zfuse(features2, 16, 128)
  t3 = _zfuse(features3, 8, 256)
  mc = mesh_coords[0]
  out = _projection_sc(t1, t2, t3, mc[:, 0], mc[:, 1], mc[:, 2],
                       mesh_features.reshape(3 * _P))
  return out[None]


# speedup vs baseline: 12.1049x; 1.0753x over previous
import functools

import jax
import jax.numpy as jnp
import numpy as np
from jax import lax
from jax.experimental import pallas as pl
from jax.experimental.pallas import tpu as pltpu
from jax.experimental.pallas import tpu_sc as plsc

_NC = 2
_NS = 16
_NW = _NC * _NS
_P = 16384
_PPW = _P // _NW
_CH = 16
_NCHUNK = _PPW // _CH
_NOUT = 451

_DNUMS = lax.GatherDimensionNumbers(
    offset_dims=(), collapsed_slice_dims=(0,), start_index_map=(0,))


def _axis_setup(c_ref, off, size):
  s = c_ref[pl.ds(off, _CH)] * np.float32(size)
  s = jnp.minimum(jnp.maximum(s, np.float32(0.01)), np.float32(size - 1.01))
  i1 = s.astype(jnp.int32)
  f1 = i1.astype(jnp.float32)
  frac = s - f1
  i2 = i1 + jnp.where(frac > np.float32(0.0), 1, 0).astype(jnp.int32)
  w_lo = i2.astype(jnp.float32) - s
  w_hi = frac
  return i1, w_lo, w_hi


def _wbcast(w, p):
  idx = jnp.full((16,), p, jnp.int32)
  return lax.gather(w, idx[:, None], dimension_numbers=_DNUMS,
                    slice_sizes=(1,),
                    mode=lax.GatherScatterMode.PROMISE_IN_BOUNDS)


def _pairs_setup(cx_v, cy_v, cz_v, off, size):
  x1, wx1, wx2 = _axis_setup(cx_v, off, size)
  y1, wy1, wy2 = _axis_setup(cy_v, off, size)
  z1, wz1, wz2 = _axis_setup(cz_v, off, size)
  x2i = x1 + 1
  y2i = y1 + 1
  pairs = ((x1, wx1, y1, wy1), (x1, wx1, y2i, wy2),
           (x2i, wx2, y1, wy1), (x2i, wx2, y2i, wy2))
  return z1, wz1, wz2, pairs


def _sc_body(t1, t2, t3, cx, cy, cz, mf, out,
             cx_v, cy_v, cz_v, mf_v, idx1, idx2, idx3,
             rows1, rows2, rows3, outv, sem1, sem2, sem3):
  wid = lax.axis_index("s") * _NC + lax.axis_index("c")
  base = wid * _PPW

  pltpu.sync_copy(cx.at[pl.ds(base, _PPW)], cx_v)
  pltpu.sync_copy(cy.at[pl.ds(base, _PPW)], cy_v)
  pltpu.sync_copy(cz.at[pl.ds(base, _PPW)], cz_v)
  pltpu.sync_copy(mf.at[pl.ds(3 * base, 3 * _PPW)],
                  mf_v.at[pl.ds(16, 3 * _PPW)])

  def fire1(i):
    off = i * _CH
    z1, _, _, pairs = _pairs_setup(cx_v, cy_v, cz_v, off, 32)
    q = z1 >> 1
    pz = z1 & np.int32(1)
    for kp, (xa, _, yb, _) in enumerate(pairs):
      strip = (xa * np.int32(32) + yb) * np.int32(16) + q
      idx1[pl.ds(kp * 32, _CH)] = strip
      idx1[pl.ds(kp * 32 + 16, _CH)] = strip + pz
    pltpu.async_copy(t1.at[idx1], rows1, sem1)

  def fire23(i, size, t_ref, idx_ref, rows_ref, sem):
    off = i * _CH
    z1, _, _, pairs = _pairs_setup(cx_v, cy_v, cz_v, off, size)
    s32 = np.int32(size)
    for kp, (xa, _, yb, _) in enumerate(pairs):
      vbase = (xa * s32 + yb) * s32 + z1
      idx_ref[pl.ds(kp * 32, _CH)] = vbase
      idx_ref[pl.ds(kp * 32 + 16, _CH)] = vbase + np.int32(1)
    pltpu.async_copy(t_ref.at[idx_ref], rows_ref, sem)

  def compute1(i):
    off = i * _CH
    z1, wz1, wz2, pairs = _pairs_setup(cx_v, cy_v, cz_v, off, 32)
    even = (z1 & np.int32(1)) == np.int32(0)
    zero = jnp.zeros((16,), jnp.float32)
    u0 = jnp.where(even, wz1, zero)
    u1 = jnp.where(even, wz2, wz1)
    u2 = jnp.where(even, zero, wz2)
    wxy = [wxa * wyb for (_, wxa, _, wyb) in pairs]

    @pl.loop(0, _CH)
    def _point(p):
      ub = [_wbcast(u, p) for u in (u0, u1, u2)]
      wb = [_wbcast(w, p) for w in wxy]
      w = [[wb[kp] * ub[s] for s in range(3)] for kp in range(4)]

      @pl.loop(0, 4, unroll=4)
      def _chanvec(j):
        lo = pl.ds(j * 16, 16)
        hi = pl.ds(64 + j * 16, 16)
        acc = (w[0][0] * rows1[p, lo] + w[0][1] * rows1[p, hi]
               + w[0][2] * rows1[_CH + p, lo])
        for kp in range(1, 4):
          r0 = kp * 32 + p
          acc = (acc + w[kp][0] * rows1[r0, lo] + w[kp][1] * rows1[r0, hi]
                 + w[kp][2] * rows1[r0 + _CH, lo])
        outv[p, lo] = acc
      outv[p, pl.ds(435, 16)] = mf_v[pl.ds(3 * (off + p) + 3, 16)]

  def compute23(i, size, col, nvec, unroll, rows_ref):
    off = i * _CH
    _, wz1, wz2, pairs = _pairs_setup(cx_v, cy_v, cz_v, off, size)
    wlo = [wxa * wyb * wz1 for (_, wxa, _, wyb) in pairs]
    whi = [wxa * wyb * wz2 for (_, wxa, _, wyb) in pairs]

    @pl.loop(0, _CH)
    def _point(p):
      ws = [(_wbcast(wlo[kp], p), _wbcast(whi[kp], p)) for kp in range(4)]

      @pl.loop(0, nvec, unroll=unroll)
      def _chanvec(j):
        sl = pl.ds(j * 16, 16)
        acc = ws[0][0] * rows_ref[p, sl] + ws[0][1] * rows_ref[_CH + p, sl]
        for kp in range(1, 4):
          r0 = kp * 32 + p
          acc = (acc + ws[kp][0] * rows_ref[r0, sl]
                 + ws[kp][1] * rows_ref[r0 + _CH, sl])
        outv[p, pl.ds(col + j * 16, 16)] = acc

  def wait(t_ref, rows_ref, sem):
    pltpu.make_async_copy(t_ref.at[pl.ds(0, rows_ref.shape[0])],
                          rows_ref, sem).wait()

  fire1(0)

  @pl.loop(0, _NCHUNK)
  def _sched(i):
    off = i * _CH
    fire23(i, 16, t2, idx2, rows2, sem2)
    wait(t1, rows1, sem1)
    compute1(i)
    fire23(i, 8, t3, idx3, rows3, sem3)
    wait(t2, rows2, sem2)
    compute23(i, 16, 64, 8, 4, rows2)

    @pl.when(i + 1 < _NCHUNK)
    def _():
      fire1(i + 1)

    wait(t3, rows3, sem3)
    compute23(i, 8, 192, 16, 4, rows3)
    pltpu.sync_copy(outv, out.at[pl.ds(base + off, _CH)])


@jax.jit
def _projection_sc(t1, t2, t3, cx, cy, cz, mf):
  mesh = plsc.VectorSubcoreMesh(core_axis_name="c", subcore_axis_name="s")
  out_type = jax.ShapeDtypeStruct((_P, _NOUT), jnp.float32)
  scratch = [
      pltpu.VMEM((_PPW,), jnp.float32),
      pltpu.VMEM((_PPW,), jnp.float32),
      pltpu.VMEM((_PPW,), jnp.float32),
      pltpu.VMEM((16 + 3 * _PPW,), jnp.float32),
      pltpu.VMEM((128,), jnp.int32),
      pltpu.VMEM((128,), jnp.int32),
      pltpu.VMEM((128,), jnp.int32),
      pltpu.VMEM((128, 128), jnp.float32),
      pltpu.VMEM((128, 128), jnp.float32),
      pltpu.VMEM((128, 256), jnp.float32),
      pltpu.VMEM((_CH, _NOUT), jnp.float32),
      pltpu.SemaphoreType.DMA,
      pltpu.SemaphoreType.DMA,
      pltpu.SemaphoreType.DMA,
  ]
  run = pl.kernel(_sc_body, out_type=out_type, mesh=mesh,
                  scratch_types=scratch)
  return run(t1, t2, t3, cx, cy, cz, mf)


def kernel(features0, features1, features2, features3, features4,
           mesh_coords, mesh_features):
  t1 = features1.reshape(16384, 128)
  t2 = features2.reshape(4096, 128)
  t3 = features3.reshape(512, 256)
  mc = mesh_coords[0]
  out = _projection_sc(t1, t2, t3, mc[:, 0], mc[:, 1], mc[:, 2],
                       mesh_features.reshape(3 * _P))
  return out[None]
